# Initial kernel scaffold; baseline (speedup 1.0000x reference)
import jax
import jax.numpy as jnp
from jax.experimental import pallas as pl


def kernel(source, target, cost_matrix):
    raise NotImplementedError("write your pallas kernel here")


# speedup vs baseline: 54.7809x; 54.7809x over previous
import jax
import jax.numpy as jnp
from jax.experimental import pallas as pl

_SPARSITY = 0.2


def _select_kernel(x_ref, o_ref, *, k):
    x = x_ref[...]
    xb = jax.lax.bitcast_convert_type(x, jnp.int32)

    def body(i, t):
        p = 29 - i
        cand = t | (1 << p)
        cnt = jnp.sum((xb < cand).astype(jnp.int32))
        return jnp.where(cnt <= k - 1, cand, t)

    t_star = jax.lax.fori_loop(0, 30, body, jnp.int32(0))
    o_ref[...] = jnp.where(xb <= t_star, x, 0.0)


def kernel(source, target, cost_matrix):
    b, n_source, n_target = cost_matrix.shape
    k = int(n_source * n_target * _SPARSITY)
    import functools

    sparse = pl.pallas_call(
        functools.partial(_select_kernel, k=k),
        grid=(b,),
        in_specs=[pl.BlockSpec((None, n_source, n_target), lambda i: (i, 0, 0))],
        out_specs=pl.BlockSpec((None, n_source, n_target), lambda i: (i, 0, 0)),
        out_shape=jax.ShapeDtypeStruct(cost_matrix.shape, cost_matrix.dtype),
    )(cost_matrix)
    return (source, target, sparse)
